# baseline (speedup 1.0000x reference)
import functools

import jax
import jax.numpy as jnp
from jax import lax
from jax.experimental import pallas as pl
from jax.experimental.pallas import tpu as pltpu
from jax.experimental.pallas import tpu_sc as plsc

D = 100
NC = 2
NS = 16
NW = NC * NS
G = 128
K = 4
CHUNK = K * G


def _make_gather(n_rows):
    rows_per_w = n_rows // NW
    ng = rows_per_w // G
    nchunk = ng // K
    assert n_rows % (NW * G) == 0 and ng % (2 * K) == 0

    mesh = plsc.VectorSubcoreMesh(core_axis_name="c", subcore_axis_name="s")

    @functools.partial(
        pl.kernel,
        out_type=jax.ShapeDtypeStruct((n_rows, D), jnp.float32),
        mesh=mesh,
        compiler_params=pltpu.CompilerParams(use_tc_tiling_on_sc=False),
        scratch_types=[
            pltpu.VMEM((K, G), jnp.int32),
            pltpu.VMEM((K, G), jnp.int32),
            pltpu.VMEM((CHUNK, D), jnp.float32),
            pltpu.VMEM((CHUNK, D), jnp.float32),
            pltpu.SemaphoreType.DMA,
            pltpu.SemaphoreType.DMA,
            pltpu.SemaphoreType.DMA,
            pltpu.SemaphoreType.DMA,
        ],
    )
    def gather_kernel(idx_hbm, table_hbm, out_hbm,
                      idx0, idx1, buf0, buf1, gsem0, gsem1, wsem0, wsem1):
        wid = lax.axis_index("s") * NC + lax.axis_index("c")
        row0 = wid * rows_per_w

        def out_slice(c):
            return out_hbm.at[pl.ds(row0 + c * CHUNK, CHUNK)]

        def idx_slice(c):
            return idx_hbm.at[wid, pl.ds(c * K, K)]

        def fire_gathers(idx_v, buf, sem):
            return [
                pltpu.async_copy(
                    table_hbm.at[idx_v.at[j]],
                    buf.at[pl.ds(j * G, G)],
                    sem,
                )
                for j in range(K)
            ]

        def body(t, _):
            a = 2 * t

            @pl.when(t > 0)
            def _drain_prev():
                pltpu.make_async_copy(buf0, out_slice(a - 2), wsem0).wait()
                pltpu.make_async_copy(buf1, out_slice(a - 1), wsem1).wait()

            pltpu.sync_copy(idx_slice(a), idx0)
            ha = fire_gathers(idx0, buf0, gsem0)
            pltpu.sync_copy(idx_slice(a + 1), idx1)
            hb = fire_gathers(idx1, buf1, gsem1)
            for h in ha:
                h.wait()
            pltpu.async_copy(buf0, out_slice(a), wsem0)
            for h in hb:
                h.wait()
            pltpu.async_copy(buf1, out_slice(a + 1), wsem1)
            return 0

        lax.fori_loop(0, nchunk // 2, body, 0)
        pltpu.make_async_copy(buf0, out_slice(nchunk - 2), wsem0).wait()
        pltpu.make_async_copy(buf1, out_slice(nchunk - 1), wsem1).wait()

    return gather_kernel


def kernel(sen, word_embeddings):
    batch, hist = sen.shape
    n_rows = batch * hist
    rows_per_w = n_rows // NW
    idx = sen.reshape(NW, rows_per_w // G, G)
    out = _make_gather(n_rows)(idx, word_embeddings)
    return out.reshape(batch, hist, D, 1)


# speedup vs baseline: 1.5560x; 1.5560x over previous
import functools

import jax
import jax.numpy as jnp
from jax import lax
from jax.experimental import pallas as pl
from jax.experimental.pallas import tpu as pltpu
from jax.experimental.pallas import tpu_sc as plsc

D = 100
DP = 128
NC = 2
NS = 16
NW = NC * NS
G = 128
K = 2
CHUNK = K * G


def _make_gather(n_rows):
    rows_per_w = n_rows // NW
    ng = rows_per_w // G
    nchunk = ng // K
    assert n_rows % (NW * G) == 0 and ng % (2 * K) == 0

    mesh = plsc.VectorSubcoreMesh(core_axis_name="c", subcore_axis_name="s")

    @functools.partial(
        pl.kernel,
        out_type=jax.ShapeDtypeStruct((n_rows, DP), jnp.float32),
        mesh=mesh,
        scratch_types=[
            pltpu.VMEM((ng, G), jnp.int32),
            pltpu.VMEM((CHUNK, DP), jnp.float32),
            pltpu.VMEM((CHUNK, DP), jnp.float32),
            pltpu.SemaphoreType.DMA,
            pltpu.SemaphoreType.DMA,
            pltpu.SemaphoreType.DMA,
            pltpu.SemaphoreType.DMA,
        ],
    )
    def gather_kernel(idx_hbm, table_hbm, out_hbm,
                      idx_v, buf0, buf1, gsem0, gsem1, wsem0, wsem1):
        wid = lax.axis_index("s") * NC + lax.axis_index("c")
        row0 = wid * rows_per_w

        pltpu.sync_copy(idx_hbm.at[wid], idx_v)

        def out_slice(c):
            return out_hbm.at[pl.ds(row0 + c * CHUNK, CHUNK)]

        def fire_gathers(c, buf, sem):
            return [
                pltpu.async_copy(
                    table_hbm.at[idx_v.at[c * K + j]],
                    buf.at[pl.ds(j * G, G)],
                    sem,
                )
                for j in range(K)
            ]

        def body(t, _):
            a = 2 * t

            @pl.when(t > 0)
            def _drain_prev():
                pltpu.make_async_copy(buf0, out_slice(a - 2), wsem0).wait()
                pltpu.make_async_copy(buf1, out_slice(a - 1), wsem1).wait()

            ha = fire_gathers(a, buf0, gsem0)
            hb = fire_gathers(a + 1, buf1, gsem1)
            for h in ha:
                h.wait()
            pltpu.async_copy(buf0, out_slice(a), wsem0)
            for h in hb:
                h.wait()
            pltpu.async_copy(buf1, out_slice(a + 1), wsem1)
            return 0

        lax.fori_loop(0, nchunk // 2, body, 0)
        pltpu.make_async_copy(buf0, out_slice(nchunk - 2), wsem0).wait()
        pltpu.make_async_copy(buf1, out_slice(nchunk - 1), wsem1).wait()

    return gather_kernel


def kernel(sen, word_embeddings):
    batch, hist = sen.shape
    n_rows = batch * hist
    rows_per_w = n_rows // NW
    idx = sen.reshape(NW, rows_per_w // G, G)
    table = jnp.pad(word_embeddings, ((0, 0), (0, DP - D)))
    out = _make_gather(n_rows)(idx, table)
    return out[:, :D].reshape(batch, hist, D, 1)
